# baseline (speedup 1.0000x reference)
import functools
import math

import jax
import jax.numpy as jnp
from jax import lax
from jax.experimental import pallas as pl
from jax.experimental.pallas import tpu as pltpu
from jax.experimental.pallas import tpu_sc as plsc

_MARGIN = 0.5
_SCALE = 64.0
_COS_M = math.cos(_MARGIN)
_SIN_M = math.sin(_MARGIN)
_OVERFLOW_THRESH = -math.cos(_MARGIN)
_C1 = _SCALE * math.log2(math.e)
_NEG_HUGE = -1e30

_ROWS_SC = 512
_CB = 2048
_PW = 1408
_NP = 71
_NCOV = _PW * _NP


def _margined(t):
    tm = t * _COS_M - jnp.sqrt(jnp.maximum(1.0 - t * t, 0.0)) * _SIN_M
    return jnp.where(t < _OVERFLOW_THRESH, t, tm)


def _sc_sumexp_body(row0, n, x_ref, y_ref, s_out, t_out,
                    x0_v, x1_v, y_v, o2_v, t2_v, tb0_v, tb1_v,
                    sem0, sem1, sem2, sem3):
    wid = lax.axis_index("s") * 2 + lax.axis_index("c")
    r0 = row0 + wid * 32
    lane = lax.iota(jnp.int32, 16)

    pltpu.sync_copy(y_ref.at[pl.ds(r0, 32)], y_v)
    y_lo = y_v[pl.ds(0, 16)]
    y_hi = y_v[pl.ds(16, 16)]

    def _copy(g, p, buf, sem):
        rg = pl.multiple_of(r0 + g * 8, 8)
        c0 = pl.multiple_of(p * _PW, 128)
        return pltpu.make_async_copy(
            x_ref.at[pl.ds(rg, 8), pl.ds(c0, _PW)], buf, sem)

    def _compute(buf):
        for r in range(8):
            acc = o2_v[r]

            def _inner(i, acc, buf=buf, r=r):
                base = i * 64
                for u in range(4):
                    x = buf[r, pl.ds(base + u * 16, 16)]
                    acc = acc + jnp.exp(x * _SCALE - _SCALE)
                return acc

            o2_v[r] = lax.fori_loop(0, _PW // 64, _inner, acc)

    def _group(g, carry):
        o2_v[...] = jnp.zeros((8, 16), jnp.float32)
        _copy(g, 0, x0_v, sem0).start()

        def _pair(q, carry):
            p0 = q * 2
            _copy(g, p0 + 1, x1_v, sem1).start()
            _copy(g, p0, x0_v, sem0).wait()
            _compute(x0_v)

            @pl.when(p0 + 2 < _NP)
            def _():
                _copy(g, p0 + 2, x0_v, sem0).start()

            _copy(g, p0 + 1, x1_v, sem1).wait()
            _compute(x1_v)
            return carry

        lax.fori_loop(0, _NP // 2, _pair, 0)
        _copy(g, _NP - 1, x0_v, sem0).wait()
        _compute(x0_v)

        rg = pl.multiple_of(r0 + g * 8, 8)
        c0t = pl.multiple_of(wid * 0 + _NCOV, 128)
        tcp = pltpu.make_async_copy(
            x_ref.at[pl.ds(rg, 8), pl.ds(c0t, 128)], tb0_v, sem2)
        tcp.start()
        tcp.wait()
        for r in range(8):
            acc = o2_v[r]
            for k in range(2):
                xx = tb0_v[r, pl.ds(k * 16, 16)]
                acc = acc + jnp.exp(xx * _SCALE - _SCALE)
            o2_v[r] = acc

        pltpu.sync_copy(
            o2_v, s_out.at[pl.ds(pl.multiple_of(wid * 32 + g * 8, 8), 8)])
        return carry

    lax.fori_loop(0, 4, _group, 0)

    tbufs = (tb0_v, tb1_v)
    tsems = (sem2, sem3)

    def _tcopy(r):
        y_r = (y_lo if r < 16 else y_hi)[r % 16]
        rg = pl.multiple_of(r0 + (r // 8) * 8, 8)
        c0 = pl.multiple_of((y_r // 128) * 128, 128)
        return pltpu.make_async_copy(
            x_ref.at[pl.ds(rg, 8), pl.ds(c0, 128)], tbufs[r % 2],
            tsems[r % 2])

    _tcopy(0).start()
    for r in range(32):
        if r + 1 < 32:
            _tcopy(r + 1).start()
        _tcopy(r).wait()
        y_r = (y_lo if r < 16 else y_hi)[r % 16]
        pos = y_r % 128
        start = pl.multiple_of((pos // 16) * 16, 16)
        v = tbufs[r % 2][r % 8, pl.ds(start, 16)]
        t2_v[r] = jnp.where(lane == pos - start, v, 0.0)
    pltpu.sync_copy(t2_v, t_out.at[pl.ds(pl.multiple_of(wid * 32, 8), 32)])


def _sc_sumexp(y_hat, y, row0, rows_sc):
    b, n = y_hat.shape
    mesh = plsc.VectorSubcoreMesh(core_axis_name="c", subcore_axis_name="s")
    kfn = functools.partial(
        pl.kernel,
        mesh=mesh,
        out_type=(
            jax.ShapeDtypeStruct((rows_sc, 16), jnp.float32),
            jax.ShapeDtypeStruct((rows_sc, 16), jnp.float32),
        ),
        scratch_types=[
            pltpu.VMEM((8, _PW), jnp.float32),
            pltpu.VMEM((8, _PW), jnp.float32),
            pltpu.VMEM((32,), jnp.int32),
            pltpu.VMEM((8, 16), jnp.float32),
            pltpu.VMEM((32, 16), jnp.float32),
            pltpu.VMEM((8, 128), jnp.float32),
            pltpu.VMEM((8, 128), jnp.float32),
            pltpu.SemaphoreType.DMA,
            pltpu.SemaphoreType.DMA,
            pltpu.SemaphoreType.DMA,
            pltpu.SemaphoreType.DMA,
        ],
    )(functools.partial(_sc_sumexp_body, row0, n))
    return kfn(y_hat, y)


def _tc_body(y_ref, x_ref, s_out, t_out, acc_ref, t_ref, *, ncb, nclass, cb):
    j = pl.program_id(0)

    @pl.when(j == 0)
    def _init():
        acc_ref[...] = jnp.zeros_like(acc_ref)
        t_ref[...] = jnp.zeros_like(t_ref)

    x = x_ref[...]
    col = j * cb + lax.broadcasted_iota(jnp.int32, x.shape, 1)
    is_t = col == y_ref[...]
    t_ref[...] = t_ref[...] + jnp.sum(jnp.where(is_t, x, 0.0), axis=1,
                                      keepdims=True)
    z = jnp.where(col < nclass, x * _C1 - _C1, _NEG_HUGE)
    acc_ref[...] = acc_ref[...] + jnp.sum(jnp.exp2(z), axis=1, keepdims=True)

    @pl.when(j == ncb - 1)
    def _fin():
        s_out[...] = acc_ref[...]
        t_out[...] = t_ref[...]


def _tc_sumexp(y_hat, y, rows_tc, cb):
    b, n = y_hat.shape
    ncb = pl.cdiv(n, cb)
    return pl.pallas_call(
        functools.partial(_tc_body, ncb=ncb, nclass=n, cb=cb),
        grid=(ncb,),
        in_specs=[
            pl.BlockSpec((rows_tc, 1), lambda j: (0, 0)),
            pl.BlockSpec((rows_tc, cb), lambda j: (0, j)),
        ],
        out_specs=(
            pl.BlockSpec((rows_tc, 1), lambda j: (0, 0)),
            pl.BlockSpec((rows_tc, 1), lambda j: (0, 0)),
        ),
        out_shape=(
            jax.ShapeDtypeStruct((rows_tc, 1), jnp.float32),
            jax.ShapeDtypeStruct((rows_tc, 1), jnp.float32),
        ),
        scratch_shapes=[
            pltpu.VMEM((rows_tc, 1), jnp.float32),
            pltpu.VMEM((rows_tc, 1), jnp.float32),
        ],
    )(y[:rows_tc].reshape(rows_tc, 1), y_hat)


def _ep_body(ssc_ref, tsc_ref, out_ref, *, batch):
    s = jnp.sum(ssc_ref[...], axis=1, keepdims=True)
    t = jnp.sum(tsc_ref[...], axis=1, keepdims=True)
    tm = _margined(t)
    e_t = jnp.exp2(t * _C1 - _C1)
    e_tm = jnp.exp2(tm * _C1 - _C1)
    s_mod = s - e_t + e_tm
    loss_rows = jnp.log(s_mod) + (_SCALE - _SCALE * tm)
    out_ref[...] = jnp.sum(loss_rows, axis=(0, 1), keepdims=True) / batch


def _epilogue(s_sc_l, t_sc_l, b):
    out = pl.pallas_call(
        functools.partial(_ep_body, batch=b),
        out_shape=jax.ShapeDtypeStruct((1, 1), jnp.float32),
    )(s_sc_l, t_sc_l)
    return out[0, 0]


def _tcr_body(y_ref, x_ref, out_ref, l_ref, *, nrb, batch):
    i = pl.program_id(0)

    @pl.when(i == 0)
    def _init():
        l_ref[...] = jnp.zeros_like(l_ref)

    x = x_ref[...]
    col = lax.broadcasted_iota(jnp.int32, x.shape, 1)
    is_t = col == y_ref[...]
    t = jnp.sum(jnp.where(is_t, x, 0.0), axis=1, keepdims=True)
    s = jnp.sum(jnp.exp2(x * _C1 - _C1), axis=1, keepdims=True)
    tm = _margined(t)
    s_mod = s - jnp.exp2(t * _C1 - _C1) + jnp.exp2(tm * _C1 - _C1)
    loss_rows = jnp.log(s_mod) + (_SCALE - _SCALE * tm)
    l_ref[...] = l_ref[...] + jnp.sum(loss_rows, axis=(0, 1), keepdims=True)

    @pl.when(i == nrb - 1)
    def _out():
        out_ref[...] = l_ref[...] / batch


def _tc_rows(y_hat, y, rb):
    b, n = y_hat.shape
    nrb = b // rb
    out = pl.pallas_call(
        functools.partial(_tcr_body, nrb=nrb, batch=b),
        grid=(nrb,),
        in_specs=[
            pl.BlockSpec((rb, 1), lambda i: (i, 0)),
            pl.BlockSpec((rb, n), lambda i: (i, 0)),
        ],
        out_specs=pl.BlockSpec((1, 1), lambda i: (0, 0)),
        out_shape=jax.ShapeDtypeStruct((1, 1), jnp.float32),
        scratch_shapes=[pltpu.VMEM((1, 1), jnp.float32)],
    )(y.reshape(b, 1), y_hat)
    return out[0, 0]


def _tcm_body(y_ref, x_hbm, out_ref, l_ref, bufs, sems, *, nch, rb, batch,
              nbuf):
    def _start(c):
        pltpu.async_copy(x_hbm.at[pl.ds(c * rb, rb), :], bufs.at[c % nbuf],
                         sems.at[c % nbuf])

    for c in range(nbuf):
        _start(c)
    l_ref[...] = jnp.zeros_like(l_ref)

    for c in range(nch):
        pltpu.make_async_copy(x_hbm.at[pl.ds(c * rb, rb), :],
                              bufs.at[c % nbuf], sems.at[c % nbuf]).wait()
        x = bufs[c % nbuf]
        col = lax.broadcasted_iota(jnp.int32, x.shape, 1)
        is_t = col == y_ref[pl.ds(c * rb, rb), :]
        t = jnp.sum(jnp.where(is_t, x, 0.0), axis=1, keepdims=True)
        s = jnp.sum(jnp.exp2(x * _C1 - _C1), axis=1, keepdims=True)
        tm = _margined(t)
        s_mod = s - jnp.exp2(t * _C1 - _C1) + jnp.exp2(tm * _C1 - _C1)
        loss_rows = jnp.log(s_mod) + (_SCALE - _SCALE * tm)
        l_ref[...] = l_ref[...] + jnp.sum(loss_rows, axis=(0, 1),
                                          keepdims=True)
        if c + nbuf < nch:
            _start(c + nbuf)

    out_ref[...] = l_ref[...] / batch


def _tc_ring(y_hat, y, rb, nbuf):
    b, n = y_hat.shape
    nch = b // rb
    out = pl.pallas_call(
        functools.partial(_tcm_body, nch=nch, rb=rb, batch=b, nbuf=nbuf),
        in_specs=[
            pl.BlockSpec((b, 1), lambda: (0, 0)),
            pl.BlockSpec(memory_space=pl.ANY),
        ],
        out_specs=pl.BlockSpec((1, 1), lambda: (0, 0)),
        out_shape=jax.ShapeDtypeStruct((1, 1), jnp.float32),
        scratch_shapes=[
            pltpu.VMEM((1, 1), jnp.float32),
            pltpu.VMEM((nbuf, rb, n), jnp.float32),
            pltpu.SemaphoreType.DMA((nbuf,)),
        ],
    )(y.reshape(b, 1), y_hat)
    return out[0, 0]


def _tcp_body(y_ref, x_ref, out_ref, l_ref, *, nrb):
    i = pl.program_id(0)

    @pl.when(i == 0)
    def _init():
        l_ref[...] = jnp.zeros_like(l_ref)

    x = x_ref[...]
    col = lax.broadcasted_iota(jnp.int32, x.shape, 1)
    is_t = col == y_ref[...]
    t = jnp.sum(jnp.where(is_t, x, 0.0), axis=1, keepdims=True)
    s = jnp.sum(jnp.exp2(x * _C1 - _C1), axis=1, keepdims=True)
    tm = _margined(t)
    s_mod = s - jnp.exp2(t * _C1 - _C1) + jnp.exp2(tm * _C1 - _C1)
    loss_rows = jnp.log(s_mod) + (_SCALE - _SCALE * tm)
    l_ref[...] = l_ref[...] + jnp.sum(loss_rows, axis=(0, 1), keepdims=True)

    @pl.when(i == nrb - 1)
    def _out():
        out_ref[...] = l_ref[...]


def _tc_rows_sum(y_hat, y, rows, rb):
    b, n = y_hat.shape
    nrb = rows // rb
    out = pl.pallas_call(
        functools.partial(_tcp_body, nrb=nrb),
        grid=(nrb,),
        in_specs=[
            pl.BlockSpec((rb, 1), lambda i: (i, 0)),
            pl.BlockSpec((rb, n), lambda i: (i, 0)),
        ],
        out_specs=pl.BlockSpec((1, 1), lambda i: (0, 0)),
        out_shape=jax.ShapeDtypeStruct((1, 1), jnp.float32),
        scratch_shapes=[pltpu.VMEM((1, 1), jnp.float32)],
    )(y[:rows].reshape(rows, 1), y_hat)
    return out


def _ep2_body(ssc_ref, tsc_ref, tcl_ref, out_ref, *, batch):
    s = jnp.sum(ssc_ref[...], axis=1, keepdims=True)
    t = jnp.sum(tsc_ref[...], axis=1, keepdims=True)
    tm = _margined(t)
    s_mod = s - jnp.exp2(t * _C1 - _C1) + jnp.exp2(tm * _C1 - _C1)
    loss_rows = jnp.log(s_mod) + (_SCALE - _SCALE * tm)
    out_ref[...] = (jnp.sum(loss_rows, axis=(0, 1), keepdims=True)
                    + tcl_ref[...]) / batch


@jax.jit
def kernel(y_hat, y):
    b, n = y_hat.shape
    half = 512
    y_sl = jax.lax.slice(y_hat, (half, 0), (b, n))
    s_sc_l, t_sc_l = _sc_sumexp(y_sl, y[half:], 0, half)
    tc_loss = _tc_rows_sum(y_hat, y, half, 64)
    out = pl.pallas_call(
        functools.partial(_ep2_body, batch=b),
        out_shape=jax.ShapeDtypeStruct((1, 1), jnp.float32),
    )(s_sc_l, t_sc_l, tc_loss)
    return out[0, 0]


# speedup vs baseline: 1.7717x; 1.7717x over previous
import functools
import math

import jax
import jax.numpy as jnp
from jax import lax
from jax.experimental import pallas as pl
from jax.experimental.pallas import tpu as pltpu

_MARGIN = 0.5
_SCALE = 64.0
_COS_M = math.cos(_MARGIN)
_SIN_M = math.sin(_MARGIN)
_OVERFLOW_THRESH = -math.cos(_MARGIN)
_C1 = _SCALE * math.log2(math.e)

_RB = 64


def _margined(t):
    tm = t * _COS_M - jnp.sqrt(jnp.maximum(1.0 - t * t, 0.0)) * _SIN_M
    return jnp.where(t < _OVERFLOW_THRESH, t, tm)


def _body(y_ref, x_ref, out_ref, l_ref, *, nrb, batch):
    i = pl.program_id(0)

    @pl.when(i == 0)
    def _init():
        l_ref[...] = jnp.zeros_like(l_ref)

    x = x_ref[...]
    col = lax.broadcasted_iota(jnp.int32, x.shape, 1)
    is_t = col == y_ref[...]
    t = jnp.sum(jnp.where(is_t, x, 0.0), axis=1, keepdims=True)
    s = jnp.sum(jnp.exp2(x * _C1 - _C1), axis=1, keepdims=True)
    tm = _margined(t)
    s_mod = s - jnp.exp2(t * _C1 - _C1) + jnp.exp2(tm * _C1 - _C1)
    loss_rows = jnp.log(s_mod) + (_SCALE - _SCALE * tm)
    l_ref[...] = l_ref[...] + jnp.sum(loss_rows, axis=(0, 1), keepdims=True)

    @pl.when(i == nrb - 1)
    def _out():
        out_ref[...] = l_ref[...] / batch


@jax.jit
def kernel(y_hat, y):
    b, n = y_hat.shape
    nrb = b // _RB
    out = pl.pallas_call(
        functools.partial(_body, nrb=nrb, batch=b),
        grid=(nrb,),
        in_specs=[
            pl.BlockSpec((_RB, 1), lambda i: (i, 0)),
            pl.BlockSpec((_RB, n), lambda i: (i, 0)),
        ],
        out_specs=pl.BlockSpec((1, 1), lambda i: (0, 0)),
        out_shape=jax.ShapeDtypeStruct((1, 1), jnp.float32),
        scratch_shapes=[pltpu.VMEM((1, 1), jnp.float32)],
    )(y.reshape(b, 1), y_hat)
    return out[0, 0]
